# baseline (speedup 1.0000x reference)
import jax
import jax.numpy as jnp
from jax.experimental import pallas as pl
from jax.experimental.pallas import tpu as pltpu


def _encoder_body(x_ref, adj_ref, w1_ref, b1_ref, th0_ref, th1_ref,
                  wt_ref, bt_ref, wr_ref, gamma_ref, beta_ref,
                  out_ref, sp_pad, featbuf):
    B, N, T, D = x_ref.shape
    F = th0_ref.shape[1]
    dot = lambda a, b: jnp.dot(a, b, preferred_element_type=jnp.float32)

    mask = (adj_ref[...] != 0.0).astype(jnp.float32)
    maskT = mask.T
    n_edges = jnp.sum(mask)
    fill = jnp.float32(N * N) - n_edges
    ii = jax.lax.broadcasted_iota(jnp.int32, (N, 1), 0)
    jj = jax.lax.broadcasted_iota(jnp.int32, (1, N), 1)
    indeg_col = jnp.sum(maskT, axis=1, keepdims=True)
    indeg_row = jnp.sum(mask, axis=0, keepdims=True)
    deg_col = jnp.maximum(indeg_col + jnp.where(ii == 0, fill, 0.0), 1.0)
    deg_row = jnp.maximum(indeg_row + jnp.where(jj == 0, fill, 0.0), 1.0)
    r_col = jax.lax.rsqrt(deg_col)
    r_row = jax.lax.rsqrt(deg_row)
    deg0 = jnp.sum(jnp.where(ii == 0, deg_col, 0.0))
    corr = fill / deg0
    ahat = maskT * (r_col * r_row)
    ahat = ahat + jnp.where((ii == 0) & (jj == 0), corr, 0.0)

    mm = jax.lax.broadcasted_iota(jnp.int32, (3 * F, 1), 0)
    nn = jax.lax.broadcasted_iota(jnp.int32, (1, F), 1)
    wtT = []
    for k in range(3):
        sel = (mm == 3 * nn + k).astype(jnp.float32)
        wtT.append(dot(wt_ref[...], sel).T)

    b1 = b1_ref[...]
    bt = bt_ref[...]
    gamma = gamma_ref[...]
    beta = beta_ref[...]

    for b in range(B):
        sp_pad[b, 0] = jnp.zeros((N, F), jnp.float32)
        sp_pad[b, T + 1] = jnp.zeros((N, F), jnp.float32)
        for t in range(T):
            feat = dot(x_ref[b, :, t, :], w1_ref[...]) + b1
            featbuf[b, t] = feat
            q0 = dot(feat, th0_ref[...])
            q1 = dot(feat, th1_ref[...])
            agg = dot(ahat, q1)
            sp_pad[b, t + 1] = jnp.maximum(q0 + agg, 0.0)

    for b in range(B):
        conv = dot(sp_pad[b, 0:T].reshape(T * N, F), wtT[0])
        conv += dot(sp_pad[b, 1:T + 1].reshape(T * N, F), wtT[1])
        conv += dot(sp_pad[b, 2:T + 2].reshape(T * N, F), wtT[2])
        res = dot(featbuf[b].reshape(T * N, D), wr_ref[...])
        out = jnp.maximum(conv + bt + res, 0.0)
        mu = jnp.mean(out, axis=1, keepdims=True)
        cent = out - mu
        var = jnp.mean(cent * cent, axis=1, keepdims=True)
        o = (cent * jax.lax.rsqrt(var + 1e-5) * gamma + beta).reshape(T, N, F)
        out_ref[b] = o


def kernel(x, adj, W1, b1, theta0, theta1, Wt, bt, Wr, ln_gamma, ln_beta):
    B, N, T, D = x.shape
    F = theta0.shape[1]
    out = pl.pallas_call(
        _encoder_body,
        out_shape=jax.ShapeDtypeStruct((B, T, N, F), jnp.float32),
        scratch_shapes=[
            pltpu.VMEM((B, T + 2, N, F), jnp.float32),
            pltpu.VMEM((B, T, N, D), jnp.float32),
        ],
    )(x, adj, W1, b1.reshape(1, -1), theta0, theta1,
      Wt.reshape(F, -1), bt.reshape(1, -1), Wr,
      ln_gamma.reshape(1, -1), ln_beta.reshape(1, -1))
    return jnp.transpose(out, (0, 2, 3, 1))


# speedup vs baseline: 1.0966x; 1.0966x over previous
import jax
import jax.numpy as jnp
from jax.experimental import pallas as pl
from jax.experimental.pallas import tpu as pltpu


def _encoder_body(x_hbm, adj_ref, w1_ref, b1_ref, th0_ref, th1_ref,
                  wt_ref, bt_ref, wr_ref, gamma_ref, beta_ref,
                  out_ref, xbuf, sp_pad, featbuf, dma_sem):
    B, N, T, D = x_hbm.shape
    F = th0_ref.shape[1]
    dot = lambda a, b: jnp.dot(a, b, preferred_element_type=jnp.float32)
    dotT = lambda a, b: jax.lax.dot_general(
        a, b, dimension_numbers=(((0,), (0,)), ((), ())),
        preferred_element_type=jnp.float32)

    copies = []
    for b in range(B):
        for t in range(T):
            c = pltpu.make_async_copy(x_hbm.at[b, :, t, :], xbuf.at[b, t],
                                      dma_sem)
            c.start()
            copies.append(c)

    mask = (adj_ref[...] != 0.0).astype(jnp.float32)
    n_edges = jnp.sum(mask)
    fill = jnp.float32(N * N) - n_edges
    ii = jax.lax.broadcasted_iota(jnp.int32, (N, 1), 0)
    jj = jax.lax.broadcasted_iota(jnp.int32, (1, N), 1)
    indeg_row = jnp.sum(mask, axis=0, keepdims=True)
    indeg_col = dotT(mask, jnp.ones((N, 1), jnp.float32))
    deg_col = jnp.maximum(indeg_col + jnp.where(ii == 0, fill, 0.0), 1.0)
    deg_row = jnp.maximum(indeg_row + jnp.where(jj == 0, fill, 0.0), 1.0)
    r_col = jax.lax.rsqrt(deg_col)
    r_row = jax.lax.rsqrt(deg_row)
    deg0 = jnp.sum(jnp.where(ii == 0, deg_col, 0.0))
    corr = fill / deg0
    ahat = mask * (r_col * r_row)
    ahat = ahat + jnp.where((ii == 0) & (jj == 0), corr, 0.0)

    mm = jax.lax.broadcasted_iota(jnp.int32, (3 * F, 1), 0)
    nn = jax.lax.broadcasted_iota(jnp.int32, (1, F), 1)
    wtT = []
    for k in range(3):
        sel = (mm == 3 * nn + k).astype(jnp.float32)
        wtT.append(dot(wt_ref[...], sel).T)

    b1 = b1_ref[...]
    bt = bt_ref[...]
    gamma = gamma_ref[...]
    beta = beta_ref[...]

    for c in copies:
        c.wait()

    for b in range(B):
        feat = dot(xbuf[b].reshape(T * N, D), w1_ref[...]) + b1
        featbuf[b] = feat.reshape(T, N, D)
        q0 = dot(feat, th0_ref[...]).reshape(T, N, F)
        q1 = dot(feat, th1_ref[...]).reshape(T, N, F)
        sp_pad[b, 0] = jnp.zeros((N, F), jnp.float32)
        sp_pad[b, T + 1] = jnp.zeros((N, F), jnp.float32)
        for t in range(T):
            agg = dotT(ahat, q1[t])
            sp_pad[b, t + 1] = jnp.maximum(q0[t] + agg, 0.0)

    for b in range(B):
        conv = dot(sp_pad[b, 0:T].reshape(T * N, F), wtT[0])
        conv += dot(sp_pad[b, 1:T + 1].reshape(T * N, F), wtT[1])
        conv += dot(sp_pad[b, 2:T + 2].reshape(T * N, F), wtT[2])
        res = dot(featbuf[b].reshape(T * N, D), wr_ref[...])
        out = jnp.maximum(conv + bt + res, 0.0)
        mu = jnp.mean(out, axis=1, keepdims=True)
        cent = out - mu
        var = jnp.mean(cent * cent, axis=1, keepdims=True)
        o = cent * jax.lax.rsqrt(var + 1e-5) * gamma + beta
        out_ref[b] = o.reshape(T, N, F)


def kernel(x, adj, W1, b1, theta0, theta1, Wt, bt, Wr, ln_gamma, ln_beta):
    B, N, T, D = x.shape
    F = theta0.shape[1]
    out = pl.pallas_call(
        _encoder_body,
        out_shape=jax.ShapeDtypeStruct((B, T, N, F), jnp.float32),
        in_specs=[pl.BlockSpec(memory_space=pltpu.HBM)] +
                 [pl.BlockSpec(memory_space=pltpu.VMEM)] * 10,
        scratch_shapes=[
            pltpu.VMEM((B, T, N, D), jnp.float32),
            pltpu.VMEM((B, T + 2, N, F), jnp.float32),
            pltpu.VMEM((B, T, N, D), jnp.float32),
            pltpu.SemaphoreType.DMA,
        ],
    )(x, adj, W1, b1.reshape(1, -1), theta0, theta1,
      Wt.reshape(F, -1), bt.reshape(1, -1), Wr,
      ln_gamma.reshape(1, -1), ln_beta.reshape(1, -1))
    return jnp.transpose(out, (0, 2, 3, 1))


# speedup vs baseline: 1.3700x; 1.2493x over previous
import jax
import jax.numpy as jnp
from jax.experimental import pallas as pl
from jax.experimental.pallas import tpu as pltpu


def _encoder_body(x_hbm, adj_ref, w1_ref, b1_ref, th0_ref, th1_ref,
                  wtT_ref, bt_ref, wr_ref, gamma_ref, beta_ref,
                  out_ref, xbuf, sp_pad, featbuf, dma_sem):
    B, N, TD = x_hbm.shape
    F = th0_ref.shape[1]
    D = w1_ref.shape[0]
    T = TD // D
    dot = lambda a, b: jnp.dot(a, b, preferred_element_type=jnp.float32)
    dotT = lambda a, b: jax.lax.dot_general(
        a, b, dimension_numbers=(((0,), (0,)), ((), ())),
        preferred_element_type=jnp.float32)

    xcopy = pltpu.make_async_copy(x_hbm, xbuf, dma_sem)
    xcopy.start()

    mask = (adj_ref[...] != 0.0).astype(jnp.float32)
    n_edges = jnp.sum(mask)
    fill = jnp.float32(N * N) - n_edges
    ii = jax.lax.broadcasted_iota(jnp.int32, (N, 1), 0)
    jj = jax.lax.broadcasted_iota(jnp.int32, (1, N), 1)
    indeg_row = jnp.sum(mask, axis=0, keepdims=True)
    indeg_col = dotT(mask, jnp.ones((N, 1), jnp.float32))
    deg_col = jnp.maximum(indeg_col + jnp.where(ii == 0, fill, 0.0), 1.0)
    deg_row = jnp.maximum(indeg_row + jnp.where(jj == 0, fill, 0.0), 1.0)
    r_col = jax.lax.rsqrt(deg_col)
    r_row = jax.lax.rsqrt(deg_row)
    deg0 = jnp.sum(jnp.where(ii == 0, deg_col, 0.0))
    corr = fill / deg0
    ahat = mask * (r_col * r_row)
    ahat = ahat + jnp.where((ii == 0) & (jj == 0), corr, 0.0)

    b1 = b1_ref[...]
    bt = bt_ref[...]
    gamma = gamma_ref[...]
    beta = beta_ref[...]

    xcopy.wait()

    for b in range(B):
        xb = xbuf[b]
        feat = jnp.concatenate(
            [xb[:, t * D:(t + 1) * D] for t in range(T)], axis=0)
        feat = dot(feat, w1_ref[...]) + b1
        featbuf[b] = feat.reshape(T, N, D)
        q0 = dot(feat, th0_ref[...]).reshape(T, N, F)
        q1 = dot(feat, th1_ref[...]).reshape(T, N, F)
        sp_pad[b, 0] = jnp.zeros((N, F), jnp.float32)
        sp_pad[b, T + 1] = jnp.zeros((N, F), jnp.float32)
        for t in range(0, T, 2):
            rhs = jnp.concatenate([q1[t], q1[t + 1]], axis=1)
            agg2 = dotT(ahat, rhs)
            sp_pad[b, t + 1] = jnp.maximum(q0[t] + agg2[:, :F], 0.0)
            sp_pad[b, t + 2] = jnp.maximum(q0[t + 1] + agg2[:, F:], 0.0)

    for b in range(B):
        conv = dot(sp_pad[b, 0:T].reshape(T * N, F), wtT_ref[0])
        conv += dot(sp_pad[b, 1:T + 1].reshape(T * N, F), wtT_ref[1])
        conv += dot(sp_pad[b, 2:T + 2].reshape(T * N, F), wtT_ref[2])
        res = dot(featbuf[b].reshape(T * N, D), wr_ref[...])
        out = jnp.maximum(conv + bt + res, 0.0)
        mu = jnp.mean(out, axis=1, keepdims=True)
        cent = out - mu
        var = jnp.mean(cent * cent, axis=1, keepdims=True)
        o = cent * jax.lax.rsqrt(var + 1e-5) * gamma + beta
        out_ref[b] = o.reshape(T, N, F)


def kernel(x, adj, W1, b1, theta0, theta1, Wt, bt, Wr, ln_gamma, ln_beta):
    B, N, T, D = x.shape
    F = theta0.shape[1]
    x2 = x.reshape(B, N, T * D)
    wtT = jnp.transpose(Wt, (2, 1, 0))
    out = pl.pallas_call(
        _encoder_body,
        out_shape=jax.ShapeDtypeStruct((B, T, N, F), jnp.float32),
        in_specs=[pl.BlockSpec(memory_space=pltpu.HBM)] +
                 [pl.BlockSpec(memory_space=pltpu.VMEM)] * 10,
        scratch_shapes=[
            pltpu.VMEM((B, N, T * D), jnp.float32),
            pltpu.VMEM((B, T + 2, N, F), jnp.float32),
            pltpu.VMEM((B, T, N, D), jnp.float32),
            pltpu.SemaphoreType.DMA,
        ],
    )(x2, adj, W1, b1.reshape(1, -1), theta0, theta1,
      wtT, bt.reshape(1, -1), Wr,
      ln_gamma.reshape(1, -1), ln_beta.reshape(1, -1))
    return jnp.transpose(out, (0, 2, 3, 1))
